# baseline (speedup 1.0000x reference)
import functools

import jax
import jax.numpy as jnp
from jax import lax
from jax.experimental import pallas as pl
from jax.experimental.pallas import tpu as pltpu
from jax.experimental.pallas import tpu_sc as plsc

N = 10000
F = 128
FH = F // 2
E = 320000
C = 80
ROWS = E // C
NW = 32
TPW = ROWS // NW
HPW = ROWS // 16
STRIPE = 624
TAIL = N - 16 * STRIPE
LG = C // 16


def _sc_agg(e3, w3, xh0, xh1):
    mesh = plsc.VectorSubcoreMesh(core_axis_name="c", subcore_axis_name="s")

    @functools.partial(
        pl.kernel,
        mesh=mesh,
        compiler_params=pltpu.CompilerParams(needs_layout_passes=False,
                                             use_tc_tiling_on_sc=False),
        out_type=jax.ShapeDtypeStruct((2, 2, N, FH), jnp.float32),
        scratch_types=[
            pltpu.VMEM((TPW, C), jnp.int32),
            pltpu.VMEM((TPW, C), jnp.int32),
            pltpu.VMEM((TPW, C), jnp.float32),
            pltpu.VMEM((N,), jnp.float32),
            pltpu.VMEM((C, FH), jnp.float32),
            pltpu.VMEM((C, FH), jnp.float32),
            pltpu.VMEM((C, FH), jnp.float32),
            pltpu.VMEM_SHARED((N,), jnp.float32),
            pltpu.VMEM_SHARED((N,), jnp.float32),
            pltpu.VMEM_SHARED((N, FH), jnp.float32),
            pltpu.SemaphoreType.DMA,
            pltpu.SemaphoreType.DMA,
            pltpu.SemaphoreType.DMA,
            pltpu.SemaphoreType.DMA,
            pltpu.SemaphoreType.DMA,
            pltpu.SemaphoreType.DMA,
        ],
    )
    def agg(e3, w3, xh0, xh1, out,
            sall, dall, nall, dinv, rows0, rows1, rows2,
            sdeg, sdinv, sacc, g0, g1, g2, s0, s1, s2):
        cid = lax.axis_index("c")
        sid = lax.axis_index("s")
        wid = cid * 16 + sid
        rows = (rows0, rows1, rows2)
        gsem = (g0, g1, g2)
        ssem = (s0, s1, s2)
        xh = (xh0, xh1)
        z16 = jnp.zeros((16,), jnp.float32)

        def zero_rows0():
            def zrow_body(r, _):
                for k in range(FH // 16):
                    rows0[r, pl.ds(k * 16, 16)] = z16
                return 0

            lax.fori_loop(0, C, zrow_body, 0)

        def zero_sacc():
            for j in range(STRIPE // C):
                pltpu.sync_copy(
                    rows0, sacc.at[pl.ds(sid * STRIPE + j * C, C), :])
            rem = STRIPE - (STRIPE // C) * C
            pltpu.sync_copy(rows0.at[pl.ds(0, rem), :],
                            sacc.at[pl.ds(sid * STRIPE + (STRIPE // C) * C,
                                          rem), :])

            @pl.when(sid == 15)
            def _():
                pltpu.sync_copy(rows0.at[pl.ds(0, TAIL), :],
                                sacc.at[pl.ds(16 * STRIPE, TAIL), :])

        zero_rows0()
        zero_sacc()

        def zdinv_body(i, _):
            dinv[pl.ds(i * 16, 16)] = z16
            return 0

        lax.fori_loop(0, (STRIPE + TAIL) // 16, zdinv_body, 0)
        pltpu.sync_copy(dinv.at[pl.ds(0, STRIPE)],
                        sdeg.at[pl.ds(sid * STRIPE, STRIPE)])

        @pl.when(sid == 15)
        def _():
            pltpu.sync_copy(dinv.at[pl.ds(0, TAIL)],
                            sdeg.at[pl.ds(16 * STRIPE, TAIL)])

        plsc.subcore_barrier()

        def mask_body(i, _):
            r = i // LG
            l = (i % LG) * 16
            s16 = sall[r, pl.ds(l, 16)]
            d16 = dall[r, pl.ds(l, 16)]
            w16 = nall[r, pl.ds(l, 16)]
            nall[r, pl.ds(l, 16)] = jnp.where(s16 == d16, 0.0, w16)
            return 0

        def deg_fire(r, _):
            pltpu.async_copy(nall.at[r], sdeg.at[sall.at[r]], g0, add=True)
            return 0

        def deg_drain(r, _):
            pltpu.make_async_copy(nall.at[r], sdeg.at[sall.at[r]], g0).wait()
            return 0

        with jax.named_scope("ph1_deg"):
            for h in range(2):
                wc = 2 * sid + h
                pltpu.sync_copy(e3.at[0, wc], sall)
                pltpu.sync_copy(e3.at[1, wc], dall)
                pltpu.sync_copy(w3.at[wc], nall)
                lax.fori_loop(0, TPW * LG, mask_body, 0)
                for bb in range(5):
                    lax.fori_loop(bb * 25, bb * 25 + 25, deg_fire, 0)
                    lax.fori_loop(bb * 25, bb * 25 + 25, deg_drain, 0)
        plsc.subcore_barrier()

        magic = jnp.full((16,), 0x5F3759DF, jnp.int32)

        def rsq_body(i, _):
            d = dinv[pl.ds(i * 16, 16)]
            yi = magic - lax.shift_right_logical(
                lax.bitcast_convert_type(d, jnp.int32), 1)
            y = lax.bitcast_convert_type(yi, jnp.float32)
            y = y * (1.5 - 0.5 * d * y * y)
            y = y * (1.5 - 0.5 * d * y * y)
            y = y * (1.5 - 0.5 * d * y * y)
            dinv[pl.ds(i * 16, 16)] = jnp.where(d > 0.0, y, 0.0)
            return 0

        with jax.named_scope("ph2_rsq"):
            pltpu.sync_copy(sdeg.at[pl.ds(sid * STRIPE, STRIPE)],
                            dinv.at[pl.ds(0, STRIPE)])

            @pl.when(sid == 15)
            def _():
                pltpu.sync_copy(sdeg.at[pl.ds(16 * STRIPE, TAIL)],
                                dinv.at[pl.ds(STRIPE, TAIL)])

            lax.fori_loop(0, STRIPE // 16, rsq_body, 0)

            @pl.when(sid == 15)
            def _():
                lax.fori_loop(STRIPE // 16, (STRIPE + TAIL) // 16,
                              rsq_body, 0)

            pltpu.sync_copy(dinv.at[pl.ds(0, STRIPE)],
                            sdinv.at[pl.ds(sid * STRIPE, STRIPE)])

            @pl.when(sid == 15)
            def _():
                pltpu.sync_copy(dinv.at[pl.ds(STRIPE, TAIL)],
                                sdinv.at[pl.ds(16 * STRIPE, TAIL)])

            plsc.subcore_barrier()
            pltpu.sync_copy(sdinv, dinv)

        def norm_body(i, _):
            r = i // LG
            l = (i % LG) * 16
            s16 = sall[r, pl.ds(l, 16)]
            d16 = dall[r, pl.ds(l, 16)]
            w16 = nall[r, pl.ds(l, 16)]
            wm = jnp.where(s16 == d16, 0.0, w16)
            gs = plsc.load_gather(dinv, [s16])
            gd = plsc.load_gather(dinv, [d16])
            nall[r, pl.ds(l, 16)] = -(gs * wm * gd)
            return 0

        with jax.named_scope("ph2b_norm"):
            pltpu.sync_copy(e3.at[0, wid], sall)
            pltpu.sync_copy(e3.at[1, wid], dall)
            pltpu.sync_copy(w3.at[wid], nall)
            lax.fori_loop(0, TPW * LG, norm_body, 0)

        def scale(rbuf, rr):
            def group_body(g, _):
                nv16 = nall[rr, pl.ds(g * 16, 16)]
                for j in range(16):
                    e = g * 16 + j
                    bc = jnp.full((16,), nv16[j], jnp.float32)
                    for k in range(FH // 16):
                        rbuf[e, pl.ds(k * 16, 16)] = (
                            rbuf[e, pl.ds(k * 16, 16)] * bc)
                return 0

            lax.fori_loop(0, LG, group_body, 0)

        def feature_pass(f):
            xf = xh[f]
            pltpu.async_copy(xf.at[sall.at[0]], rows0, g0)
            pltpu.async_copy(xf.at[sall.at[1]], rows1, g1)

            def triple_body(q, _):
                for X in range(3):
                    c = 3 * q + X
                    Y = (X + 2) % 3

                    @pl.when(c + 2 <= TPW - 1)
                    def _():
                        @pl.when(c >= 1)
                        def _():
                            pltpu.make_async_copy(
                                rows[Y], sacc.at[dall.at[c - 1]],
                                ssem[Y]).wait()

                        pltpu.async_copy(xf.at[sall.at[c + 2]], rows[Y],
                                         gsem[Y])

                    @pl.when(c <= TPW - 1)
                    def _():
                        pltpu.make_async_copy(xf.at[sall.at[c]], rows[X],
                                              gsem[X]).wait()
                        scale(rows[X], c)
                        pltpu.async_copy(rows[X], sacc.at[dall.at[c]],
                                         ssem[X], add=True)

                return 0

            lax.fori_loop(0, (TPW + 2) // 3, triple_body, 0)
            for c in (TPW - 3, TPW - 2, TPW - 1):
                pltpu.make_async_copy(rows[c % 3], sacc.at[dall.at[c]],
                                      ssem[c % 3]).wait()
            plsc.subcore_barrier()
            pltpu.sync_copy(sacc.at[pl.ds(sid * STRIPE, STRIPE), :],
                            out.at[cid, f, pl.ds(sid * STRIPE, STRIPE), :])

            @pl.when(sid == 15)
            def _():
                pltpu.sync_copy(sacc.at[pl.ds(16 * STRIPE, TAIL), :],
                                out.at[cid, f, pl.ds(16 * STRIPE, TAIL), :])

        with jax.named_scope("ph3_msg"):
            feature_pass(0)
            plsc.subcore_barrier()
            zero_rows0()
            zero_sacc()
            plsc.subcore_barrier()
            feature_pass(1)

    return agg(e3, w3, xh0, xh1)


def _tc_body(x_ref, p00_ref, p01_ref, p10_ref, p11_ref, wzh_ref, bzh_ref,
             wlt_ref, blin_ref, o_ref):
    tl = p00_ref[...] + p10_ref[...]
    tr = p01_ref[...] + p11_ref[...]
    xx = x_ref[...]
    logits = jnp.dot(xx, wzh_ref[0:F, :], preferred_element_type=jnp.float32)
    logits = logits + jnp.dot(tl, wzh_ref[F:F + FH, :],
                              preferred_element_type=jnp.float32)
    logits = logits + jnp.dot(tr, wzh_ref[F + FH:2 * F, :],
                              preferred_element_type=jnp.float32)
    logits = logits + bzh_ref[...]
    z = jax.nn.sigmoid(logits[:, 0:F])
    ht = jnp.tanh(logits[:, F:2 * F])
    h = jnp.maximum((1.0 - z) * ht, 0.0)
    o_ref[...] = jnp.dot(h, wlt_ref[...],
                         preferred_element_type=jnp.float32) + blin_ref[...]


def _tc_dense(x, p00, p01, p10, p11, wzh, bzh, wlt, blin):
    R = 2000
    grid = (N // R,)
    return pl.pallas_call(
        _tc_body,
        grid=grid,
        in_specs=[
            pl.BlockSpec((R, F), lambda i: (i, 0)),
            pl.BlockSpec((R, FH), lambda i: (i, 0)),
            pl.BlockSpec((R, FH), lambda i: (i, 0)),
            pl.BlockSpec((R, FH), lambda i: (i, 0)),
            pl.BlockSpec((R, FH), lambda i: (i, 0)),
            pl.BlockSpec((2 * F, 2 * F), lambda i: (0, 0)),
            pl.BlockSpec((1, 2 * F), lambda i: (0, 0)),
            pl.BlockSpec((F, F), lambda i: (0, 0)),
            pl.BlockSpec((1, F), lambda i: (0, 0)),
        ],
        out_specs=pl.BlockSpec((R, F), lambda i: (i, 0)),
        out_shape=jax.ShapeDtypeStruct((N, F), jnp.float32),
    )(x, p00, p01, p10, p11, wzh, bzh, wlt, blin)


def kernel(x, edge_index, edge_weight, W_xz, b_xz, W_hz, b_hz, W_xr, b_xr,
           W_hr, b_hr, W_xh, b_xh, W_hh, b_hh, W_lin, b_lin):
    e3 = edge_index.astype(jnp.int32).reshape(2, NW, TPW, C)
    w3 = edge_weight.astype(jnp.float32).reshape(NW, TPW, C)
    xh0 = x[:, :FH]
    xh1 = x[:, FH:]
    tx4 = _sc_agg(e3, w3, xh0, xh1)

    wzh = jnp.concatenate([
        jnp.concatenate([W_xz[0], W_xh[0]], axis=1),
        jnp.concatenate([W_xz[1], W_xh[1]], axis=1),
    ], axis=0)
    bzh = jnp.concatenate([b_xz + b_hz, b_xh + b_hh]).reshape(1, 2 * F)
    wlt = W_lin.T
    blin = b_lin.reshape(1, F)
    return _tc_dense(x, tx4[0, 0], tx4[0, 1], tx4[1, 0], tx4[1, 1],
                     wzh, bzh, wlt, blin)


# speedup vs baseline: 2.1112x; 2.1112x over previous
import functools

import jax
import jax.numpy as jnp
from jax import lax
from jax.experimental import pallas as pl
from jax.experimental.pallas import tpu as pltpu
from jax.experimental.pallas import tpu_sc as plsc

N = 10000
F = 128
E = 320000
C = 80
ROWS = E // C
NW = 32
TPW = ROWS // NW
NB = 5
B = TPW // NB
STRIPE = 624
TAIL = N - 16 * STRIPE
LG = C // 16


def _sc_agg(e4, w4, x):
    mesh = plsc.VectorSubcoreMesh(core_axis_name="c", subcore_axis_name="s")

    @functools.partial(
        pl.kernel,
        mesh=mesh,
        compiler_params=pltpu.CompilerParams(needs_layout_passes=False),
        out_type=[jax.ShapeDtypeStruct((2, N, F), jnp.float32),
                  jax.ShapeDtypeStruct((2, N, F), jnp.float32)],
        scratch_types=[
            pltpu.VMEM((B, C), jnp.int32),
            pltpu.VMEM((B, C), jnp.int32),
            pltpu.VMEM((B, C), jnp.float32),
            pltpu.VMEM((STRIPE + TAIL, ), jnp.float32),
            pltpu.VMEM((C, F), jnp.float32),
            pltpu.VMEM((C, F), jnp.float32),
            pltpu.VMEM((C, F), jnp.float32),
            pltpu.VMEM_SHARED((N,), jnp.float32),
            pltpu.VMEM_SHARED((N, F), jnp.float32),
            pltpu.SemaphoreType.DMA,
            pltpu.SemaphoreType.DMA,
            pltpu.SemaphoreType.DMA,
            pltpu.SemaphoreType.DMA,
            pltpu.SemaphoreType.DMA,
            pltpu.SemaphoreType.DMA,
        ],
    )
    def agg(e4, w4, x, out, y,
            sblk, dblk, nblk, dstr, rows0, rows1, rows2,
            sdeg, sacc, g0, g1, g2, s0, s1, s2):
        cid = lax.axis_index("c")
        sid = lax.axis_index("s")
        wid = cid * 16 + sid
        rows = (rows0, rows1, rows2)
        gsem = (g0, g1, g2)
        ssem = (s0, s1, s2)
        z16 = jnp.zeros((16,), jnp.float32)
        base = sid * STRIPE

        def zrow_body(r, _):
            for k in range(8):
                rows0[r, pl.ds(k * 16, 16)] = z16
            return 0

        lax.fori_loop(0, C, zrow_body, 0)

        def zstr_body(i, _):
            dstr[pl.ds(i * 16, 16)] = z16
            return 0

        lax.fori_loop(0, (STRIPE + TAIL) // 16, zstr_body, 0)

        for j in range(STRIPE // C):
            pltpu.sync_copy(rows0, sacc.at[pl.ds(base + j * C, C), :])
        REM = STRIPE - (STRIPE // C) * C
        pltpu.sync_copy(rows0.at[pl.ds(0, REM), :],
                        sacc.at[pl.ds(base + (STRIPE // C) * C, REM), :])
        pltpu.sync_copy(dstr.at[pl.ds(0, STRIPE)],
                        sdeg.at[pl.ds(base, STRIPE)])

        @pl.when(sid == 15)
        def _():
            pltpu.sync_copy(rows0.at[pl.ds(0, TAIL), :],
                            sacc.at[pl.ds(16 * STRIPE, TAIL), :])
            pltpu.sync_copy(dstr.at[pl.ds(0, TAIL)],
                            sdeg.at[pl.ds(16 * STRIPE, TAIL)])

        plsc.subcore_barrier()

        def mask_body(i, _):
            r = i // LG
            l = (i % LG) * 16
            s16 = sblk[r, pl.ds(l, 16)]
            d16 = dblk[r, pl.ds(l, 16)]
            w16 = nblk[r, pl.ds(l, 16)]
            nblk[r, pl.ds(l, 16)] = jnp.where(s16 == d16, 0.0, w16)
            return 0

        def deg_fire(r, _):
            pltpu.async_copy(nblk.at[r], sdeg.at[sblk.at[r]], g0, add=True)
            return 0

        def deg_drain(r, _):
            pltpu.make_async_copy(nblk.at[r], sdeg.at[sblk.at[r]], g0).wait()
            return 0

        def deg_block(hb, _):
            h = hb // NB
            b = hb % NB
            pltpu.sync_copy(e4.at[0, 2 * sid + h, b], sblk)
            pltpu.sync_copy(e4.at[1, 2 * sid + h, b], dblk)
            pltpu.sync_copy(w4.at[2 * sid + h, b], nblk)
            lax.fori_loop(0, B * LG, mask_body, 0)
            lax.fori_loop(0, B, deg_fire, 0)
            lax.fori_loop(0, B, deg_drain, 0)
            return 0

        with jax.named_scope("ph1_deg"):
            lax.fori_loop(0, 2 * NB, deg_block, 0)
        plsc.subcore_barrier()

        magic = jnp.full((16,), 0x5F3759DF, jnp.int32)

        def rsq_body(i, _):
            d = dstr[pl.ds(i * 16, 16)]
            yi = magic - lax.shift_right_logical(
                lax.bitcast_convert_type(d, jnp.int32), 1)
            yv = lax.bitcast_convert_type(yi, jnp.float32)
            yv = yv * (1.5 - 0.5 * d * yv * yv)
            yv = yv * (1.5 - 0.5 * d * yv * yv)
            yv = yv * (1.5 - 0.5 * d * yv * yv)
            dstr[pl.ds(i * 16, 16)] = jnp.where(d > 0.0, yv, 0.0)
            return 0

        with jax.named_scope("ph2_rsq"):
            pltpu.sync_copy(sdeg.at[pl.ds(base, STRIPE)],
                            dstr.at[pl.ds(0, STRIPE)])

            @pl.when(sid == 15)
            def _():
                pltpu.sync_copy(sdeg.at[pl.ds(16 * STRIPE, TAIL)],
                                dstr.at[pl.ds(STRIPE, TAIL)])

            lax.fori_loop(0, STRIPE // 16, rsq_body, 0)

            @pl.when(sid == 15)
            def _():
                lax.fori_loop(STRIPE // 16, (STRIPE + TAIL) // 16,
                              rsq_body, 0)

        def stripe_scale(nrows, src_off, buf_off, neg):
            def srow_body(i, _):
                nv16 = dstr[pl.ds(src_off + i * 16, 16)]

                def one(j):
                    e = buf_off + i * 16 + j
                    v = nv16[j]
                    bc = jnp.full((16,), -v if neg else v, jnp.float32)
                    for k in range(8):
                        rows0[e, pl.ds(k * 16, 16)] = (
                            rows0[e, pl.ds(k * 16, 16)] * bc)

                for j in range(16):
                    one(j)
                return 0

            lax.fori_loop(0, nrows // 16, srow_body, 0)

        def stripe_pass(src_hbm, dst_hbm, neg):
            for g in range(STRIPE // C):
                pltpu.sync_copy(src_hbm.at[pl.ds(base + g * C, C), :], rows0)
                stripe_scale(C, g * C, 0, neg)
                pltpu.sync_copy(rows0, dst_hbm.at[pl.ds(base + g * C, C), :])
            go = (STRIPE // C) * C
            pltpu.sync_copy(src_hbm.at[pl.ds(base + go, REM), :],
                            rows0.at[pl.ds(0, REM), :])
            stripe_scale(REM, go, 0, neg)
            pltpu.sync_copy(rows0.at[pl.ds(0, REM), :],
                            dst_hbm.at[pl.ds(base + go, REM), :])

            @pl.when(sid == 15)
            def _():
                pltpu.sync_copy(src_hbm.at[pl.ds(16 * STRIPE, TAIL), :],
                                rows0.at[pl.ds(0, TAIL), :])
                stripe_scale(TAIL, STRIPE, 0, neg)
                pltpu.sync_copy(rows0.at[pl.ds(0, TAIL), :],
                                dst_hbm.at[pl.ds(16 * STRIPE, TAIL), :])

        with jax.named_scope("ph2b_y"):
            stripe_pass(x, y.at[cid], neg=False)
        plsc.subcore_barrier()

        def scale(rbuf, rr):
            def group_body(g, _):
                nv16 = nblk[rr, pl.ds(g * 16, 16)]
                for j in range(16):
                    e = g * 16 + j
                    bc = jnp.full((16,), nv16[j], jnp.float32)
                    for k in range(8):
                        rbuf[e, pl.ds(k * 16, 16)] = (
                            rbuf[e, pl.ds(k * 16, 16)] * bc)
                return 0

            lax.fori_loop(0, LG, group_body, 0)

        yc = y.at[cid]

        def msg_block(b, _):
            pltpu.sync_copy(e4.at[0, wid, b], sblk)
            pltpu.sync_copy(e4.at[1, wid, b], dblk)
            pltpu.sync_copy(w4.at[wid, b], nblk)
            lax.fori_loop(0, B * LG, mask_body, 0)

            pltpu.async_copy(yc.at[sblk.at[0]], rows0, g0)
            pltpu.async_copy(yc.at[sblk.at[1]], rows1, g1)

            def triple_body(q, _):
                for X in range(3):
                    c = 3 * q + X
                    Y = (X + 2) % 3

                    @pl.when(c + 2 <= B - 1)
                    def _():
                        @pl.when(c >= 1)
                        def _():
                            pltpu.make_async_copy(
                                rows[Y], sacc.at[dblk.at[c - 1]],
                                ssem[Y]).wait()

                        pltpu.async_copy(yc.at[sblk.at[c + 2]], rows[Y],
                                         gsem[Y])

                    @pl.when(c <= B - 1)
                    def _():
                        pltpu.make_async_copy(yc.at[sblk.at[c]], rows[X],
                                              gsem[X]).wait()
                        scale(rows[X], c)
                        pltpu.async_copy(rows[X], sacc.at[dblk.at[c]],
                                         ssem[X], add=True)

                return 0

            lax.fori_loop(0, (B + 2) // 3, triple_body, 0)
            for c in (B - 3, B - 2, B - 1):
                pltpu.make_async_copy(rows[c % 3], sacc.at[dblk.at[c]],
                                      ssem[c % 3]).wait()
            return 0

        with jax.named_scope("ph3_msg"):
            lax.fori_loop(0, NB, msg_block, 0)
        plsc.subcore_barrier()

        with jax.named_scope("ph4_wb"):
            stripe_pass(sacc, out.at[cid], neg=True)

    return agg(e4, w4, x)


def _tc_body(x_ref, p0_ref, p1_ref, wzh_ref, bzh_ref, wlt_ref, blin_ref,
             o_ref):
    t = p0_ref[...] + p1_ref[...]
    xx = x_ref[...]
    logits = jnp.dot(xx, wzh_ref[0:F, :], preferred_element_type=jnp.float32)
    logits = logits + jnp.dot(t, wzh_ref[F:2 * F, :],
                              preferred_element_type=jnp.float32)
    logits = logits + bzh_ref[...]
    z = jax.nn.sigmoid(logits[:, 0:F])
    ht = jnp.tanh(logits[:, F:2 * F])
    h = jnp.maximum((1.0 - z) * ht, 0.0)
    o_ref[...] = jnp.dot(h, wlt_ref[...],
                         preferred_element_type=jnp.float32) + blin_ref[...]


def _tc_dense(x, p0, p1, wzh, bzh, wlt, blin):
    R = 2000
    grid = (N // R,)
    return pl.pallas_call(
        _tc_body,
        grid=grid,
        in_specs=[
            pl.BlockSpec((R, F), lambda i: (i, 0)),
            pl.BlockSpec((R, F), lambda i: (i, 0)),
            pl.BlockSpec((R, F), lambda i: (i, 0)),
            pl.BlockSpec((2 * F, 2 * F), lambda i: (0, 0)),
            pl.BlockSpec((1, 2 * F), lambda i: (0, 0)),
            pl.BlockSpec((F, F), lambda i: (0, 0)),
            pl.BlockSpec((1, F), lambda i: (0, 0)),
        ],
        out_specs=pl.BlockSpec((R, F), lambda i: (i, 0)),
        out_shape=jax.ShapeDtypeStruct((N, F), jnp.float32),
    )(x, p0, p1, wzh, bzh, wlt, blin)


def kernel(x, edge_index, edge_weight, W_xz, b_xz, W_hz, b_hz, W_xr, b_xr,
           W_hr, b_hr, W_xh, b_xh, W_hh, b_hh, W_lin, b_lin):
    e4 = edge_index.astype(jnp.int32).reshape(2, NW, NB, B, C)
    w4 = edge_weight.astype(jnp.float32).reshape(NW, NB, B, C)
    tx1p, _ = _sc_agg(e4, w4, x)

    wzh = jnp.concatenate([
        jnp.concatenate([W_xz[0], W_xh[0]], axis=1),
        jnp.concatenate([W_xz[1], W_xh[1]], axis=1),
    ], axis=0)
    bzh = jnp.concatenate([b_xz + b_hz, b_xh + b_hh]).reshape(1, 2 * F)
    wlt = W_lin.T
    blin = b_lin.reshape(1, F)
    return _tc_dense(x, tx1p[0], tx1p[1], wzh, bzh, wlt, blin)


# speedup vs baseline: 2.1992x; 1.0417x over previous
import functools

import jax
import jax.numpy as jnp
from jax import lax
from jax.experimental import pallas as pl
from jax.experimental.pallas import tpu as pltpu
from jax.experimental.pallas import tpu_sc as plsc

N = 10000
F = 128
E = 320000
C = 80
ROWS = E // C
NW = 32
TPW = ROWS // NW
NB = 5
B = TPW // NB
STRIPE = 624
TAIL = N - 16 * STRIPE
LG = C // 16


def _sc_agg(e4, w4, x):
    mesh = plsc.VectorSubcoreMesh(core_axis_name="c", subcore_axis_name="s")

    @functools.partial(
        pl.kernel,
        mesh=mesh,
        compiler_params=pltpu.CompilerParams(needs_layout_passes=False),
        out_type=[jax.ShapeDtypeStruct((2, N, F), jnp.float32),
                  jax.ShapeDtypeStruct((2, N, F), jnp.float32)],
        scratch_types=[
            pltpu.VMEM((B, C), jnp.int32),
            pltpu.VMEM((B, C), jnp.int32),
            pltpu.VMEM((B, C), jnp.float32),
            pltpu.VMEM((STRIPE + TAIL, ), jnp.float32),
            pltpu.VMEM((C, F), jnp.float32),
            pltpu.VMEM((C, F), jnp.float32),
            pltpu.VMEM((C, F), jnp.float32),
            pltpu.VMEM_SHARED((N,), jnp.float32),
            pltpu.VMEM_SHARED((N, F), jnp.float32),
            pltpu.SemaphoreType.DMA,
            pltpu.SemaphoreType.DMA,
            pltpu.SemaphoreType.DMA,
            pltpu.SemaphoreType.DMA,
            pltpu.SemaphoreType.DMA,
            pltpu.SemaphoreType.DMA,
        ],
    )
    def agg(e4, w4, x, out, y,
            sblk, dblk, nblk, dstr, rows0, rows1, rows2,
            sdeg, sacc, g0, g1, g2, s0, s1, s2):
        cid = lax.axis_index("c")
        sid = lax.axis_index("s")
        wid = cid * 16 + sid
        rows = (rows0, rows1, rows2)
        gsem = (g0, g1, g2)
        ssem = (s0, s1, s2)
        z16 = jnp.zeros((16,), jnp.float32)
        base = sid * STRIPE

        def zrow_body(r, _):
            for k in range(8):
                rows0[r, pl.ds(k * 16, 16)] = z16
            return 0

        lax.fori_loop(0, C, zrow_body, 0)

        def zstr_body(i, _):
            dstr[pl.ds(i * 16, 16)] = z16
            return 0

        lax.fori_loop(0, (STRIPE + TAIL) // 16, zstr_body, 0)

        REM = STRIPE - (STRIPE // C) * C
        for j in range(STRIPE // C):
            pltpu.async_copy(rows0, sacc.at[pl.ds(base + j * C, C), :], g0)
        pltpu.async_copy(rows0.at[pl.ds(0, REM), :],
                         sacc.at[pl.ds(base + (STRIPE // C) * C, REM), :], g1)
        pltpu.sync_copy(dstr.at[pl.ds(0, STRIPE)],
                        sdeg.at[pl.ds(base, STRIPE)])
        for j in range(STRIPE // C):
            pltpu.make_async_copy(
                rows0, sacc.at[pl.ds(base + j * C, C), :], g0).wait()
        pltpu.make_async_copy(
            rows0.at[pl.ds(0, REM), :],
            sacc.at[pl.ds(base + (STRIPE // C) * C, REM), :], g1).wait()

        @pl.when(sid == 15)
        def _():
            pltpu.sync_copy(rows0.at[pl.ds(0, TAIL), :],
                            sacc.at[pl.ds(16 * STRIPE, TAIL), :])
            pltpu.sync_copy(dstr.at[pl.ds(0, TAIL)],
                            sdeg.at[pl.ds(16 * STRIPE, TAIL)])

        plsc.subcore_barrier()

        def mask_body(i, _):
            r = i // LG
            l = (i % LG) * 16
            s16 = sblk[r, pl.ds(l, 16)]
            d16 = dblk[r, pl.ds(l, 16)]
            w16 = nblk[r, pl.ds(l, 16)]
            nblk[r, pl.ds(l, 16)] = jnp.where(s16 == d16, 0.0, w16)
            return 0

        def deg_fire(r, _):
            pltpu.async_copy(nblk.at[r], sdeg.at[sblk.at[r]], g0, add=True)
            return 0

        def deg_drain(r, _):
            pltpu.make_async_copy(nblk.at[r], sdeg.at[sblk.at[r]], g0).wait()
            return 0

        def deg_block(hb, _):
            h = hb // NB
            b = hb % NB
            pltpu.sync_copy(e4.at[0, 2 * sid + h, b], sblk)
            pltpu.sync_copy(e4.at[1, 2 * sid + h, b], dblk)
            pltpu.sync_copy(w4.at[2 * sid + h, b], nblk)
            lax.fori_loop(0, B * LG, mask_body, 0)
            lax.fori_loop(0, B, deg_fire, 0)
            lax.fori_loop(0, B, deg_drain, 0)
            return 0

        with jax.named_scope("ph1_deg"):
            lax.fori_loop(0, 2 * NB, deg_block, 0)
        plsc.subcore_barrier()

        magic = jnp.full((16,), 0x5F3759DF, jnp.int32)

        def rsq_body(i, _):
            d = dstr[pl.ds(i * 16, 16)]
            yi = magic - lax.shift_right_logical(
                lax.bitcast_convert_type(d, jnp.int32), 1)
            yv = lax.bitcast_convert_type(yi, jnp.float32)
            yv = yv * (1.5 - 0.5 * d * yv * yv)
            yv = yv * (1.5 - 0.5 * d * yv * yv)
            yv = yv * (1.5 - 0.5 * d * yv * yv)
            dstr[pl.ds(i * 16, 16)] = jnp.where(d > 0.0, yv, 0.0)
            return 0

        with jax.named_scope("ph2_rsq"):
            pltpu.sync_copy(sdeg.at[pl.ds(base, STRIPE)],
                            dstr.at[pl.ds(0, STRIPE)])

            @pl.when(sid == 15)
            def _():
                pltpu.sync_copy(sdeg.at[pl.ds(16 * STRIPE, TAIL)],
                                dstr.at[pl.ds(STRIPE, TAIL)])

            lax.fori_loop(0, STRIPE // 16, rsq_body, 0)

            @pl.when(sid == 15)
            def _():
                lax.fori_loop(STRIPE // 16, (STRIPE + TAIL) // 16,
                              rsq_body, 0)

        def stripe_scale(buf, nrows, src_off, neg):
            def srow_body(i, _):
                nv16 = dstr[pl.ds(src_off + i * 16, 16)]
                for j in range(16):
                    e = i * 16 + j
                    v = nv16[j]
                    bc = jnp.full((16,), -v if neg else v, jnp.float32)
                    for k in range(8):
                        buf[e, pl.ds(k * 16, 16)] = (
                            buf[e, pl.ds(k * 16, 16)] * bc)
                return 0

            lax.fori_loop(0, nrows // 16, srow_body, 0)

        def stripe_pass(src_hbm, dst_hbm, neg):
            NG = STRIPE // C
            sizes = [C] * NG + [REM]
            offs = [g * C for g in range(NG)] + [NG * C]

            def gin(g):
                return pltpu.make_async_copy(
                    src_hbm.at[pl.ds(base + offs[g], sizes[g]), :],
                    rows[g % 2].at[pl.ds(0, sizes[g]), :], gsem[g % 2])

            def gout(g):
                return pltpu.make_async_copy(
                    rows[g % 2].at[pl.ds(0, sizes[g]), :],
                    dst_hbm.at[pl.ds(base + offs[g], sizes[g]), :],
                    ssem[g % 2])

            gin(0).start()
            for g in range(NG + 1):
                if g + 1 <= NG:
                    if g >= 1:
                        gout(g - 1).wait()
                    gin(g + 1).start()
                gin(g).wait()
                stripe_scale(rows[g % 2], sizes[g], offs[g], neg)
                gout(g).start()
            gout(NG - 1).wait()
            gout(NG).wait()

            @pl.when(sid == 15)
            def _():
                pltpu.sync_copy(src_hbm.at[pl.ds(16 * STRIPE, TAIL), :],
                                rows0.at[pl.ds(0, TAIL), :])
                stripe_scale(rows0, TAIL, STRIPE, neg)
                pltpu.sync_copy(rows0.at[pl.ds(0, TAIL), :],
                                dst_hbm.at[pl.ds(16 * STRIPE, TAIL), :])

        with jax.named_scope("ph2b_y"):
            stripe_pass(x, y.at[cid], neg=False)
        plsc.subcore_barrier()

        def scale(rbuf, rr):
            def group_body(g, _):
                nv16 = nblk[rr, pl.ds(g * 16, 16)]
                for j in range(16):
                    e = g * 16 + j
                    bc = jnp.full((16,), nv16[j], jnp.float32)
                    for k in range(8):
                        rbuf[e, pl.ds(k * 16, 16)] = (
                            rbuf[e, pl.ds(k * 16, 16)] * bc)
                return 0

            lax.fori_loop(0, LG, group_body, 0)

        yc = y.at[cid]

        def msg_block(b, _):
            pltpu.sync_copy(e4.at[0, wid, b], sblk)
            pltpu.sync_copy(e4.at[1, wid, b], dblk)
            pltpu.sync_copy(w4.at[wid, b], nblk)
            lax.fori_loop(0, B * LG, mask_body, 0)

            pltpu.async_copy(yc.at[sblk.at[0]], rows0, g0)
            pltpu.async_copy(yc.at[sblk.at[1]], rows1, g1)

            def triple_body(q, _):
                for X in range(3):
                    c = 3 * q + X
                    Y = (X + 2) % 3

                    @pl.when(c + 2 <= B - 1)
                    def _():
                        @pl.when(c >= 1)
                        def _():
                            pltpu.make_async_copy(
                                rows[Y], sacc.at[dblk.at[c - 1]],
                                ssem[Y]).wait()

                        pltpu.async_copy(yc.at[sblk.at[c + 2]], rows[Y],
                                         gsem[Y])

                    @pl.when(c <= B - 1)
                    def _():
                        pltpu.make_async_copy(yc.at[sblk.at[c]], rows[X],
                                              gsem[X]).wait()
                        scale(rows[X], c)
                        pltpu.async_copy(rows[X], sacc.at[dblk.at[c]],
                                         ssem[X], add=True)

                return 0

            lax.fori_loop(0, (B + 2) // 3, triple_body, 0)
            for c in (B - 3, B - 2, B - 1):
                pltpu.make_async_copy(rows[c % 3], sacc.at[dblk.at[c]],
                                      ssem[c % 3]).wait()
            return 0

        with jax.named_scope("ph3_msg"):
            lax.fori_loop(0, NB, msg_block, 0)
        plsc.subcore_barrier()

        with jax.named_scope("ph4_wb"):
            stripe_pass(sacc, out.at[cid], neg=True)

    return agg(e4, w4, x)


def _tc_body(x_ref, p0_ref, p1_ref, wzh_ref, bzh_ref, wlt_ref, blin_ref,
             o_ref):
    t = p0_ref[...] + p1_ref[...]
    xx = x_ref[...]
    logits = jnp.dot(xx, wzh_ref[0:F, :], preferred_element_type=jnp.float32)
    logits = logits + jnp.dot(t, wzh_ref[F:2 * F, :],
                              preferred_element_type=jnp.float32)
    logits = logits + bzh_ref[...]
    z = jax.nn.sigmoid(logits[:, 0:F])
    ht = jnp.tanh(logits[:, F:2 * F])
    h = jnp.maximum((1.0 - z) * ht, 0.0)
    o_ref[...] = jnp.dot(h, wlt_ref[...],
                         preferred_element_type=jnp.float32) + blin_ref[...]


def _tc_dense(x, p0, p1, wzh, bzh, wlt, blin):
    R = 2000
    grid = (N // R,)
    return pl.pallas_call(
        _tc_body,
        grid=grid,
        in_specs=[
            pl.BlockSpec((R, F), lambda i: (i, 0)),
            pl.BlockSpec((R, F), lambda i: (i, 0)),
            pl.BlockSpec((R, F), lambda i: (i, 0)),
            pl.BlockSpec((2 * F, 2 * F), lambda i: (0, 0)),
            pl.BlockSpec((1, 2 * F), lambda i: (0, 0)),
            pl.BlockSpec((F, F), lambda i: (0, 0)),
            pl.BlockSpec((1, F), lambda i: (0, 0)),
        ],
        out_specs=pl.BlockSpec((R, F), lambda i: (i, 0)),
        out_shape=jax.ShapeDtypeStruct((N, F), jnp.float32),
    )(x, p0, p1, wzh, bzh, wlt, blin)


def kernel(x, edge_index, edge_weight, W_xz, b_xz, W_hz, b_hz, W_xr, b_xr,
           W_hr, b_hr, W_xh, b_xh, W_hh, b_hh, W_lin, b_lin):
    e4 = edge_index.astype(jnp.int32).reshape(2, NW, NB, B, C)
    w4 = edge_weight.astype(jnp.float32).reshape(NW, NB, B, C)
    tx1p, _ = _sc_agg(e4, w4, x)

    wzh = jnp.concatenate([
        jnp.concatenate([W_xz[0], W_xh[0]], axis=1),
        jnp.concatenate([W_xz[1], W_xh[1]], axis=1),
    ], axis=0)
    bzh = jnp.concatenate([b_xz + b_hz, b_xh + b_hh]).reshape(1, 2 * F)
    wlt = W_lin.T
    blin = b_lin.reshape(1, F)
    return _tc_dense(x, tx1p[0], tx1p[1], wzh, bzh, wlt, blin)
